# Initial kernel scaffold; baseline (speedup 1.0000x reference)
import functools

import jax
import jax.numpy as jnp
from jax import lax
from jax.experimental import pallas as pl
from jax.experimental.pallas import tpu as pltpu
from jax.experimental.pallas import tpu_sc as plsc

N = 10000
E = 320000
D = 128
G = 10

NC = 2
NS = 16
C = 128
NCHUNKS = E // C
CPC = NCHUNKS // NC
ITERS = (CPC + NS - 1) // NS
RPT = N // NS
ZR = 125

_mesh = plsc.VectorSubcoreMesh(core_axis_name="c", subcore_axis_name="s")


@functools.partial(
    pl.kernel,
    out_type=jax.ShapeDtypeStruct((NC, N, D), jnp.float32),
    mesh=_mesh,
    scratch_types=[
        pltpu.VMEM((C,), jnp.int32),
        pltpu.VMEM((C,), jnp.int32),
        pltpu.VMEM((C, D), jnp.float32),
        pltpu.VMEM((ZR, D), jnp.float32),
        pltpu.VMEM_SHARED((N, D), jnp.float32),
        pltpu.SemaphoreType.DMA,
    ],
)
def _sc_agg(h_hbm, src_hbm, dst_hbm, out_hbm, sidx, didx, rows, zbuf, acc, sem):
    c = lax.axis_index("c")
    s = lax.axis_index("s")

    zv = jnp.zeros((16,), jnp.float32)

    def _zrow(r, carry):
        for j in range(D // 16):
            zbuf[r, pl.ds(j * 16, 16)] = zv
        return carry

    lax.fori_loop(0, ZR, _zrow, 0)

    for k in range(RPT // ZR):
        pltpu.sync_copy(zbuf, acc.at[pl.ds(s * RPT + k * ZR, ZR)])
    plsc.subcore_barrier()

    def _chunk(i, carry):
        local = s + NS * i

        @pl.when(local < CPC)
        def _():
            chunk = c * CPC + local
            pltpu.sync_copy(src_hbm.at[chunk], sidx)
            pltpu.sync_copy(dst_hbm.at[chunk], didx)
            pltpu.async_copy(h_hbm.at[sidx], rows, sem).wait()
            pltpu.sync_copy(rows, acc.at[didx], add=True)

        return carry

    lax.fori_loop(0, ITERS, _chunk, 0)
    plsc.subcore_barrier()

    pltpu.sync_copy(acc.at[pl.ds(s * RPT, RPT)],
                    out_hbm.at[c, pl.ds(s * RPT, RPT)])


RB = 2000
PB = N // G


def _mid_body(x_ref, a_ref, w1_ref, b1_ref, w2_ref, b2_ref, o_ref):
    z = x_ref[...] + a_ref[0] + a_ref[1]
    z = jnp.dot(z, w1_ref[...], preferred_element_type=jnp.float32) + b1_ref[...]
    z = jnp.maximum(z, 0.0)
    z = jnp.dot(z, w2_ref[...], preferred_element_type=jnp.float32) + b2_ref[...]
    o_ref[...] = jnp.maximum(z, 0.0)


def _mlp_mid(h, agg, W1, b1, W2, b2):
    return pl.pallas_call(
        _mid_body,
        grid=(N // RB,),
        in_specs=[
            pl.BlockSpec((RB, D), lambda g: (g, 0)),
            pl.BlockSpec((NC, RB, D), lambda g: (0, g, 0)),
            pl.BlockSpec((D, D), lambda g: (0, 0)),
            pl.BlockSpec((1, D), lambda g: (0, 0)),
            pl.BlockSpec((D, D), lambda g: (0, 0)),
            pl.BlockSpec((1, D), lambda g: (0, 0)),
        ],
        out_specs=pl.BlockSpec((RB, D), lambda g: (g, 0)),
        out_shape=jax.ShapeDtypeStruct((N, D), jnp.float32),
    )(h, agg, W1, b1.reshape(1, D), W2, b2.reshape(1, D))


def _pool_body(x_ref, a_ref, w1_ref, b1_ref, w2_ref, b2_ref, o_ref):
    z = x_ref[...] + a_ref[0] + a_ref[1]
    z = jnp.dot(z, w1_ref[...], preferred_element_type=jnp.float32) + b1_ref[...]
    z = jnp.maximum(z, 0.0)
    y = jnp.dot(z, w2_ref[...], preferred_element_type=jnp.float32) + b2_ref[...]
    o_ref[...] = jnp.sum(y, axis=0, keepdims=True) * (1.0 / PB)


def _mlp_pool(h, agg, W1, b1, W2, b2):
    return pl.pallas_call(
        _pool_body,
        grid=(G,),
        in_specs=[
            pl.BlockSpec((PB, D), lambda g: (g, 0)),
            pl.BlockSpec((NC, PB, D), lambda g: (0, g, 0)),
            pl.BlockSpec((D, D), lambda g: (0, 0)),
            pl.BlockSpec((1, D), lambda g: (0, 0)),
            pl.BlockSpec((D, D), lambda g: (0, 0)),
            pl.BlockSpec((1, D), lambda g: (0, 0)),
        ],
        out_specs=pl.BlockSpec((1, D), lambda g: (g, 0)),
        out_shape=jax.ShapeDtypeStruct((G, D), jnp.float32),
    )(h, agg, W1, b1.reshape(1, D), W2, b2.reshape(1, D))


def kernel(x, edge_index, ptr, W1_0, b1_0, W2_0, b2_0, W1_1, b1_1, W2_1, b2_1):
    src = edge_index[0].reshape(NCHUNKS, C)
    dst = edge_index[1].reshape(NCHUNKS, C)
    agg0 = _sc_agg(x, src, dst)
    h1 = _mlp_mid(x, agg0, W1_0, b1_0, W2_0, b2_0)
    agg1 = _sc_agg(h1, src, dst)
    return _mlp_pool(h1, agg1, W1_1, b1_1, W2_1, b2_1)


# speedup vs baseline: 6.6255x; 6.6255x over previous
import functools

import jax
import jax.numpy as jnp
from jax import lax
from jax.experimental import pallas as pl
from jax.experimental.pallas import tpu as pltpu
from jax.experimental.pallas import tpu_sc as plsc

N = 10000
E = 320000
D = 128
G = 10

NC = 2
NS = 16
C = 128
NCHUNKS = E // C
CPC = NCHUNKS // NC
ITERS = (CPC + NS - 1) // NS
WB = 624
WREM = N - NS * WB
ZR = 208

def _sc_agg_body(h_hbm, src_hbm, dst_hbm, out_hbm, sidx, didx, rows, zbuf, acc, sem):
    c = lax.axis_index("c")
    s = lax.axis_index("s")

    zv = jnp.zeros((16,), jnp.float32)

    def _zrow(r, carry):
        for j in range(D // 16):
            zbuf[r, pl.ds(j * 16, 16)] = zv
        return carry

    lax.fori_loop(0, ZR, _zrow, 0)

    for k in range(WB // ZR):
        pltpu.sync_copy(zbuf, acc.at[pl.ds(s * WB + k * ZR, ZR)])

    @pl.when(s == NS - 1)
    def _zrem():
        pltpu.sync_copy(zbuf.at[pl.ds(0, WREM)], acc.at[pl.ds(NS * WB, WREM)])

    plsc.subcore_barrier()

    def _chunk(i, carry):
        local = s + NS * i

        @pl.when(local < CPC)
        def _():
            chunk = c * CPC + local
            pltpu.sync_copy(src_hbm.at[chunk], sidx)
            pltpu.sync_copy(dst_hbm.at[chunk], didx)
            pltpu.async_copy(h_hbm.at[sidx.at[0]], rows, sem).wait()
            pltpu.sync_copy(rows, acc.at[didx.at[0]], add=True)

        return carry

    lax.fori_loop(0, ITERS, _chunk, 0)
    plsc.subcore_barrier()

    pltpu.sync_copy(acc.at[pl.ds(s * WB, WB)],
                    out_hbm.at[c, pl.ds(s * WB, WB)])

    @pl.when(s == NS - 1)
    def _wrem():
        pltpu.sync_copy(acc.at[pl.ds(NS * WB, WREM)],
                        out_hbm.at[c, pl.ds(NS * WB, WREM)])


@functools.cache
def _get_sc_agg():
    mesh = plsc.VectorSubcoreMesh(core_axis_name="c", subcore_axis_name="s",
                                  num_cores=NC, num_subcores=NS)
    return pl.kernel(
        _sc_agg_body,
        out_type=jax.ShapeDtypeStruct((NC, N, D), jnp.float32),
        mesh=mesh,
        scratch_types=[
            pltpu.VMEM((1, C), jnp.int32),
            pltpu.VMEM((1, C), jnp.int32),
            pltpu.VMEM((C, D), jnp.float32),
            pltpu.VMEM((ZR, D), jnp.float32),
            pltpu.VMEM_SHARED((N, D), jnp.float32),
            pltpu.SemaphoreType.DMA,
        ],
    )


RB = 2000
PB = N // G


def _mid_body(x_ref, a_ref, w1_ref, b1_ref, w2_ref, b2_ref, o_ref):
    z = x_ref[...] + a_ref[0] + a_ref[1]
    z = jnp.dot(z, w1_ref[...], preferred_element_type=jnp.float32) + b1_ref[...]
    z = jnp.maximum(z, 0.0)
    z = jnp.dot(z, w2_ref[...], preferred_element_type=jnp.float32) + b2_ref[...]
    o_ref[...] = jnp.maximum(z, 0.0)


def _mlp_mid(h, agg, W1, b1, W2, b2):
    return pl.pallas_call(
        _mid_body,
        grid=(N // RB,),
        in_specs=[
            pl.BlockSpec((RB, D), lambda g: (g, 0)),
            pl.BlockSpec((NC, RB, D), lambda g: (0, g, 0)),
            pl.BlockSpec((D, D), lambda g: (0, 0)),
            pl.BlockSpec((1, D), lambda g: (0, 0)),
            pl.BlockSpec((D, D), lambda g: (0, 0)),
            pl.BlockSpec((1, D), lambda g: (0, 0)),
        ],
        out_specs=pl.BlockSpec((RB, D), lambda g: (g, 0)),
        out_shape=jax.ShapeDtypeStruct((N, D), jnp.float32),
    )(h, agg, W1, b1.reshape(1, D), W2, b2.reshape(1, D))


def _pool_body(x_ref, a_ref, w1_ref, b1_ref, w2_ref, b2_ref, o_ref):
    z = x_ref[...] + a_ref[0] + a_ref[1]
    z = jnp.dot(z, w1_ref[...], preferred_element_type=jnp.float32) + b1_ref[...]
    z = jnp.maximum(z, 0.0)
    y = jnp.dot(z, w2_ref[...], preferred_element_type=jnp.float32) + b2_ref[...]
    o_ref[0] = jnp.sum(y, axis=0, keepdims=True) * (1.0 / PB)


def _mlp_pool(h, agg, W1, b1, W2, b2):
    return pl.pallas_call(
        _pool_body,
        grid=(G,),
        in_specs=[
            pl.BlockSpec((PB, D), lambda g: (g, 0)),
            pl.BlockSpec((NC, PB, D), lambda g: (0, g, 0)),
            pl.BlockSpec((D, D), lambda g: (0, 0)),
            pl.BlockSpec((1, D), lambda g: (0, 0)),
            pl.BlockSpec((D, D), lambda g: (0, 0)),
            pl.BlockSpec((1, D), lambda g: (0, 0)),
        ],
        out_specs=pl.BlockSpec((1, 1, D), lambda g: (g, 0, 0)),
        out_shape=jax.ShapeDtypeStruct((G, 1, D), jnp.float32),
    )(h, agg, W1, b1.reshape(1, D), W2, b2.reshape(1, D)).reshape(G, D)


def kernel(x, edge_index, ptr, W1_0, b1_0, W2_0, b2_0, W1_1, b1_1, W2_1, b2_1):
    src = edge_index[0].reshape(NCHUNKS, 1, C)
    dst = edge_index[1].reshape(NCHUNKS, 1, C)
    sc_agg = _get_sc_agg()
    agg0 = sc_agg(x, src, dst)
    h1 = _mlp_mid(x, agg0, W1_0, b1_0, W2_0, b2_0)
    agg1 = sc_agg(h1, src, dst)
    return _mlp_pool(h1, agg1, W1_1, b1_1, W2_1, b2_1)


# speedup vs baseline: 13.3825x; 2.0198x over previous
import functools

import jax
import jax.numpy as jnp
from jax import lax
from jax.experimental import pallas as pl
from jax.experimental.pallas import tpu as pltpu
from jax.experimental.pallas import tpu_sc as plsc

N = 10000
E = 320000
D = 128
G = 10

NC = 2
NS = 16
C = 128
NCHUNKS = E // C
CPC = NCHUNKS // NC
CPT = 78
NBUF = 2
IGRP = 6
NGRP = CPT // IGRP
IROWS = 2 * IGRP
NREM = CPC - NS * CPT
WB = 624
WREM = N - NS * WB
ZR = 104

def _sc_agg_body(h_hbm, src_hbm, dst_hbm, out_hbm, sidx, didx, rows, zbuf, acc,
                 isem, gsems, ssems):
    c = lax.axis_index("c")
    s = lax.axis_index("s")

    zv = jnp.zeros((16,), jnp.float32)

    def _zrow(r, carry):
        for j in range(D // 16):
            zbuf[r, pl.ds(j * 16, 16)] = zv
        return carry

    lax.fori_loop(0, ZR, _zrow, 0)

    for k in range(WB // ZR):
        pltpu.sync_copy(zbuf, acc.at[pl.ds(s * WB + k * ZR, ZR)])

    @pl.when(s == NS - 1)
    def _zrem():
        pltpu.sync_copy(zbuf.at[pl.ds(0, WREM)], acc.at[pl.ds(NS * WB, WREM)])

    plsc.subcore_barrier()

    base = c * CPC + s * CPT

    def _idx_load(grp, slot):
        off = pl.ds(slot * IGRP, IGRP)
        pltpu.async_copy(src_hbm.at[pl.ds(base + grp * IGRP, IGRP)],
                         sidx.at[off], isem)
        pltpu.async_copy(dst_hbm.at[pl.ds(base + grp * IGRP, IGRP)],
                         didx.at[off], isem)

    def _idx_wait():
        for _ in range(2):
            pltpu.make_async_copy(src_hbm.at[pl.ds(0, IGRP)],
                                  sidx.at[pl.ds(0, IGRP)], isem).wait()

    def _gather(j, b):
        pltpu.async_copy(h_hbm.at[sidx.at[lax.rem(j, IROWS), 0]],
                         rows.at[b], gsems[b])

    def _gather_wait(b):
        pltpu.make_async_copy(h_hbm.at[pl.ds(0, C)], rows.at[b],
                              gsems[b]).wait()

    def _scatter(j, b):
        return pltpu.async_copy(rows.at[b],
                                acc.at[didx.at[lax.rem(j, IROWS), 0]],
                                ssems[b], add=True)

    _idx_load(0, 0)
    _idx_wait()
    _idx_load(1, 1)
    for b in range(NBUF):
        _gather(b, b)

    def _outer(g, carry):
        @pl.when(g + 1 < NGRP)
        def _():
            _idx_wait()

        for k in range(IGRP):
            j = g * IGRP + k
            b = k % NBUF
            _gather_wait(b)
            _scatter(j, b).wait()
            jn = j + NBUF

            @pl.when(jn < CPT)
            def _():
                _gather(jn, b)

        @pl.when(g + 2 < NGRP)
        def _():
            _idx_load(g + 2, lax.rem(g, 2))

        return carry

    lax.fori_loop(0, NGRP, _outer, 0)

    @pl.when(s < NREM)
    def _rem():
        chunk = c * CPC + NS * CPT + s
        pltpu.sync_copy(src_hbm.at[chunk], sidx.at[0])
        pltpu.sync_copy(dst_hbm.at[chunk], didx.at[0])
        pltpu.async_copy(h_hbm.at[sidx.at[0, 0]], rows.at[0], gsems[0]).wait()
        pltpu.sync_copy(rows.at[0], acc.at[didx.at[0, 0]], add=True)

    plsc.subcore_barrier()

    pltpu.sync_copy(acc.at[pl.ds(s * WB, WB)],
                    out_hbm.at[c, pl.ds(s * WB, WB)])

    @pl.when(s == NS - 1)
    def _wrem():
        pltpu.sync_copy(acc.at[pl.ds(NS * WB, WREM)],
                        out_hbm.at[c, pl.ds(NS * WB, WREM)])


@functools.cache
def _get_sc_agg():
    mesh = plsc.VectorSubcoreMesh(core_axis_name="c", subcore_axis_name="s",
                                  num_cores=NC, num_subcores=NS)
    return pl.kernel(
        _sc_agg_body,
        out_type=jax.ShapeDtypeStruct((NC, N, D), jnp.float32),
        mesh=mesh,
        scratch_types=[
            pltpu.VMEM((IROWS, 1, C), jnp.int32),
            pltpu.VMEM((IROWS, 1, C), jnp.int32),
            pltpu.VMEM((NBUF, C, D), jnp.float32),
            pltpu.VMEM((ZR, D), jnp.float32),
            pltpu.VMEM_SHARED((N, D), jnp.float32),
            pltpu.SemaphoreType.DMA,
            [pltpu.SemaphoreType.DMA] * NBUF,
            [pltpu.SemaphoreType.DMA] * NBUF,
        ],
    )


RB = 2000
PB = N // G


def _mid_body(x_ref, a_ref, w1_ref, b1_ref, w2_ref, b2_ref, o_ref):
    z = x_ref[...] + a_ref[0] + a_ref[1]
    z = jnp.dot(z, w1_ref[...], preferred_element_type=jnp.float32) + b1_ref[...]
    z = jnp.maximum(z, 0.0)
    z = jnp.dot(z, w2_ref[...], preferred_element_type=jnp.float32) + b2_ref[...]
    o_ref[...] = jnp.maximum(z, 0.0)


def _mlp_mid(h, agg, W1, b1, W2, b2):
    return pl.pallas_call(
        _mid_body,
        grid=(N // RB,),
        in_specs=[
            pl.BlockSpec((RB, D), lambda g: (g, 0)),
            pl.BlockSpec((NC, RB, D), lambda g: (0, g, 0)),
            pl.BlockSpec((D, D), lambda g: (0, 0)),
            pl.BlockSpec((1, D), lambda g: (0, 0)),
            pl.BlockSpec((D, D), lambda g: (0, 0)),
            pl.BlockSpec((1, D), lambda g: (0, 0)),
        ],
        out_specs=pl.BlockSpec((RB, D), lambda g: (g, 0)),
        out_shape=jax.ShapeDtypeStruct((N, D), jnp.float32),
    )(h, agg, W1, b1.reshape(1, D), W2, b2.reshape(1, D))


def _pool_body(x_ref, a_ref, w1_ref, b1_ref, w2_ref, b2_ref, o_ref):
    z = x_ref[...] + a_ref[0] + a_ref[1]
    z = jnp.dot(z, w1_ref[...], preferred_element_type=jnp.float32) + b1_ref[...]
    z = jnp.maximum(z, 0.0)
    y = jnp.dot(z, w2_ref[...], preferred_element_type=jnp.float32) + b2_ref[...]
    o_ref[0] = jnp.sum(y, axis=0, keepdims=True) * (1.0 / PB)


def _mlp_pool(h, agg, W1, b1, W2, b2):
    return pl.pallas_call(
        _pool_body,
        grid=(G,),
        in_specs=[
            pl.BlockSpec((PB, D), lambda g: (g, 0)),
            pl.BlockSpec((NC, PB, D), lambda g: (0, g, 0)),
            pl.BlockSpec((D, D), lambda g: (0, 0)),
            pl.BlockSpec((1, D), lambda g: (0, 0)),
            pl.BlockSpec((D, D), lambda g: (0, 0)),
            pl.BlockSpec((1, D), lambda g: (0, 0)),
        ],
        out_specs=pl.BlockSpec((1, 1, D), lambda g: (g, 0, 0)),
        out_shape=jax.ShapeDtypeStruct((G, 1, D), jnp.float32),
    )(h, agg, W1, b1.reshape(1, D), W2, b2.reshape(1, D)).reshape(G, D)


def kernel(x, edge_index, ptr, W1_0, b1_0, W2_0, b2_0, W1_1, b1_1, W2_1, b2_1):
    src = edge_index[0].reshape(NCHUNKS, 1, C)
    dst = edge_index[1].reshape(NCHUNKS, 1, C)
    sc_agg = _get_sc_agg()
    agg0 = sc_agg(x, src, dst)
    h1 = _mlp_mid(x, agg0, W1_0, b1_0, W2_0, b2_0)
    agg1 = sc_agg(h1, src, dst)
    return _mlp_pool(h1, agg1, W1_1, b1_1, W2_1, b2_1)
